# baseline (speedup 1.0000x reference)
import jax
import jax.numpy as jnp
from jax.experimental import pallas as pl
from jax.experimental.pallas import tpu as pltpu

GAMMA = 0.1
TAO = 10.0
B_CONST = 1.0
BETA = 1.0
LAMBDA_ = 0.1
EPS = 1e-06

K = 8192
D = 256
CR = 2048
NCH = K // CR
NBUF = 4
PREISSUE = 4


def _softplus(z):
    return jnp.log1p(jnp.exp(z))


def _body(label_ref, c_ref, lab_ref, p_hbm, loss_ref, mind_ref,
          buf, d2s, sems):
    copies = [None] * NCH

    def chunk_copy(i):
        return pltpu.make_async_copy(
            p_hbm.at[pl.ds(i * CR, CR)], buf.at[i % NBUF], sems.at[i % NBUF])

    for i in range(PREISSUE):
        copies[i] = chunk_copy(i)
        copies[i].start()

    ce = c_ref[...] + EPS
    ones = jnp.ones((D, 1), jnp.float32)
    for i in range(NCH):
        copies[i].wait()
        if i + PREISSUE < NCH:
            copies[i + PREISSUE] = chunk_copy(i + PREISSUE)
            copies[i + PREISSUE].start()
        diff = ce - buf[i % NBUF]
        sq = diff * diff
        d2s[pl.ds(i * (CR // 128), CR // 128), :] = jax.lax.dot_general(
            sq, ones, (((1,), (0,)), ((), ())),
            preferred_element_type=jnp.float32).reshape(CR // 128, 128)

    d2 = d2s[...]
    mask = lab_ref[...] == label_ref[0, 0]
    d = jnp.sqrt(d2)
    e = jnp.exp(-GAMMA * d2)
    one = jnp.sum(e)
    num = jnp.sum(jnp.where(mask, e, 0.0))
    g1 = _softplus(B_CONST - (TAO - d))
    g2 = _softplus(B_CONST + (TAO - d))
    pw = jnp.sum(jnp.where(mask, g1, 0.0)) + jnp.sum(g2)
    mind2 = jnp.min(jnp.where(mask, d2, jnp.inf))
    dce = -jnp.log(num / one)
    loss_ref[0, 0] = dce + LAMBDA_ * pw
    mind_ref[0, 0] = jnp.sqrt(mind2)


_tc_full = pl.pallas_call(
    _body,
    in_specs=[
        pl.BlockSpec(memory_space=pltpu.SMEM),
        pl.BlockSpec(memory_space=pltpu.VMEM),
        pl.BlockSpec(memory_space=pltpu.VMEM),
        pl.BlockSpec(memory_space=pl.ANY),
    ],
    out_specs=(
        pl.BlockSpec(memory_space=pltpu.SMEM),
        pl.BlockSpec(memory_space=pltpu.SMEM),
    ),
    out_shape=(
        jax.ShapeDtypeStruct((1, 1), jnp.float32),
        jax.ShapeDtypeStruct((1, 1), jnp.float32),
    ),
    scratch_shapes=[
        pltpu.VMEM((NBUF, CR, D), jnp.float32),
        pltpu.VMEM((K // 128, 128), jnp.float32),
        pltpu.SemaphoreType.DMA((NBUF,)),
    ],
)


def kernel(feature, label, prototypes, proto_labels):
    lab = proto_labels.astype(jnp.int32).reshape(K // 128, 128)
    label2d = jnp.asarray(label, jnp.int32).reshape(1, 1)
    loss, mind = _tc_full(label2d, feature.astype(jnp.float32), lab, prototypes)
    return (loss.reshape(()), mind.reshape(()))


# speedup vs baseline: 1.0136x; 1.0136x over previous
import jax
import jax.numpy as jnp
from jax.experimental import pallas as pl
from jax.experimental.pallas import tpu as pltpu

GAMMA = 0.1
TAO = 10.0
B_CONST = 1.0
BETA = 1.0
LAMBDA_ = 0.1
EPS = 1e-06

K = 8192
D = 256
SIZES = (512, 1536, 2048, 2048, 2048)
OFFS = (0, 512, 2048, 4096, 6144)
NCH = len(SIZES)


def _softplus(z):
    return jnp.log1p(jnp.exp(z))


def _body(label_ref, c_ref, lab_ref, p_hbm, loss_ref, mind_ref,
          stage, d2s, sems):
    copies = []
    for i in range(NCH):
        cp = pltpu.make_async_copy(
            p_hbm.at[pl.ds(OFFS[i], SIZES[i])],
            stage.at[pl.ds(OFFS[i], SIZES[i])], sems.at[i])
        cp.start()
        copies.append(cp)

    ce = c_ref[...] + EPS
    ones = jnp.ones((D, 1), jnp.float32)
    for i in range(NCH):
        copies[i].wait()
        diff = ce - stage[pl.ds(OFFS[i], SIZES[i]), :]
        sq = diff * diff
        d2s[pl.ds(OFFS[i] // 128, SIZES[i] // 128), :] = jax.lax.dot_general(
            sq, ones, (((1,), (0,)), ((), ())),
            preferred_element_type=jnp.float32).reshape(SIZES[i] // 128, 128)

    d2 = d2s[...]
    mask = lab_ref[...] == label_ref[0, 0]
    d = jnp.sqrt(d2)
    e = jnp.exp(-GAMMA * d2)
    one = jnp.sum(e)
    num = jnp.sum(jnp.where(mask, e, 0.0))
    g1 = _softplus(B_CONST - (TAO - d))
    g2 = _softplus(B_CONST + (TAO - d))
    pw = jnp.sum(jnp.where(mask, g1, 0.0)) + jnp.sum(g2)
    mind2 = jnp.min(jnp.where(mask, d2, jnp.inf))
    dce = -jnp.log(num / one)
    loss_ref[0, 0] = dce + LAMBDA_ * pw
    mind_ref[0, 0] = jnp.sqrt(mind2)


_tc_full = pl.pallas_call(
    _body,
    in_specs=[
        pl.BlockSpec(memory_space=pltpu.SMEM),
        pl.BlockSpec(memory_space=pltpu.VMEM),
        pl.BlockSpec(memory_space=pltpu.VMEM),
        pl.BlockSpec(memory_space=pl.ANY),
    ],
    out_specs=(
        pl.BlockSpec(memory_space=pltpu.SMEM),
        pl.BlockSpec(memory_space=pltpu.SMEM),
    ),
    out_shape=(
        jax.ShapeDtypeStruct((1, 1), jnp.float32),
        jax.ShapeDtypeStruct((1, 1), jnp.float32),
    ),
    scratch_shapes=[
        pltpu.VMEM((K, D), jnp.float32),
        pltpu.VMEM((K // 128, 128), jnp.float32),
        pltpu.SemaphoreType.DMA((NCH,)),
    ],
)


def kernel(feature, label, prototypes, proto_labels):
    lab = proto_labels.astype(jnp.int32).reshape(K // 128, 128)
    label2d = jnp.asarray(label, jnp.int32).reshape(1, 1)
    loss, mind = _tc_full(label2d, feature.astype(jnp.float32), lab, prototypes)
    return (loss.reshape(()), mind.reshape(()))
